# baseline (speedup 1.0000x reference)
import functools

import jax
import jax.numpy as jnp
from jax.experimental import pallas as pl
from jax.experimental.pallas import tpu as pltpu


def _tc_body(tab_ref, lg_ref, ct_ref, o_ref):
    x = lg_ref[...]
    idx = jnp.minimum(ct_ref[...], 10)
    r = jnp.full_like(x, 0.0) + tab_ref[0, 0]
    for t in range(1, 11):
        r = jnp.where(idx >= t, tab_ref[0, t], r)
    m = tab_ref[0, 11]
    o_ref[...] = m * jnp.tanh(x * r)


def kernel(logits, alt_counts, confidence, max_logit):
    n = logits.shape[0]
    rows, cols = 1000, 1000
    br = 200
    m = max_logit.astype(jnp.float32)
    tab = jnp.zeros((1, 16), jnp.float32).at[0, :11].set(confidence / m)
    tab = tab.at[0, 11].set(m)
    lg2 = logits.reshape(rows, cols)
    ct2 = alt_counts.reshape(rows, cols)
    grid = rows // br
    out = pl.pallas_call(
        _tc_body,
        grid=(grid,),
        in_specs=[
            pl.BlockSpec((1, 16), lambda i: (0, 0)),
            pl.BlockSpec((br, cols), lambda i: (i, 0)),
            pl.BlockSpec((br, cols), lambda i: (i, 0)),
        ],
        out_specs=pl.BlockSpec((br, cols), lambda i: (i, 0)),
        out_shape=jax.ShapeDtypeStruct((rows, cols), jnp.float32),
    )(tab, lg2, ct2)
    return out.reshape(n)


# speedup vs baseline: 1.1354x; 1.1354x over previous
import functools

import jax
import jax.numpy as jnp
from jax import lax
from jax.experimental import pallas as pl
from jax.experimental.pallas import tpu as pltpu
from jax.experimental.pallas import tpu_sc as plsc

_L = 16
_NC, _NS = 2, 16
_NW = _NC * _NS
_UNROLL = 8
_NCHUNK = 5

_N_TC = 614400
_BS_TC = 122880


def _sc_run_factory(n, n_sc, k):
    off = n - n_sc
    q = _UNROLL * _NCHUNK
    nv = -(-(n_sc // _L) // _NW)
    nv = -(-nv // q) * q
    ch = nv * _L
    cnv = nv // _NCHUNK
    cch = cnv * _L
    kmax = k - 1

    mesh = plsc.VectorSubcoreMesh(core_axis_name="c", subcore_axis_name="s")

    @functools.partial(
        pl.kernel,
        out_type=jax.ShapeDtypeStruct((n_sc,), jnp.float32),
        mesh=mesh,
        compiler_params=pltpu.CompilerParams(needs_layout_passes=False),
        scratch_types=[
            pltpu.VMEM((cch,), jnp.float32),
            pltpu.VMEM((cch,), jnp.float32),
            pltpu.VMEM((cch,), jnp.int32),
            pltpu.VMEM((cch,), jnp.int32),
            pltpu.VMEM((cch,), jnp.float32),
            pltpu.VMEM((cch,), jnp.float32),
            pltpu.VMEM((3 * _L,), jnp.float32),
            pltpu.SemaphoreType.DMA,
            pltpu.SemaphoreType.DMA,
            pltpu.SemaphoreType.DMA,
            pltpu.SemaphoreType.DMA,
            pltpu.SemaphoreType.DMA,
            pltpu.SemaphoreType.DMA,
        ],
    )
    def run(logits_hbm, counts_hbm, params_hbm, out_hbm,
            lg0, lg1, ct0, ct1, o0, o1, par_v,
            slg0, slg1, sct0, sct1, sout0, sout1):
        lg_b = (lg0, lg1)
        ct_b = (ct0, ct1)
        out_b = (o0, o1)
        slg = (slg0, slg1)
        sct = (sct0, sct1)
        sout = (sout0, sout1)
        wid = lax.axis_index("s") * _NC + lax.axis_index("c")
        base = jnp.minimum(wid * ch, n_sc - ch)
        pltpu.sync_copy(params_hbm, par_v)
        tabr = par_v.at[pl.ds(0, _L)]
        pmv = par_v[pl.ds(_L, _L)]
        p2mv = par_v[pl.ds(2 * _L, _L)]

        def start_in(j):
            b = j % 2
            src = off + base + j * cch
            hl = pltpu.async_copy(
                logits_hbm.at[pl.ds(src, cch)], lg_b[b], slg[b])
            hc = pltpu.async_copy(
                counts_hbm.at[pl.ds(src, cch)], ct_b[b], sct[b])
            return hl, hc

        hin = [None] * _NCHUNK
        hout = [None] * _NCHUNK
        hin[0] = start_in(0)
        for j in range(_NCHUNK):
            if j + 1 < _NCHUNK:
                hin[j + 1] = start_in(j + 1)
            hin[j][0].wait()
            hin[j][1].wait()
            if j >= 2:
                hout[j - 2].wait()
            b = j % 2
            lgb, ctb, outb = lg_b[b], ct_b[b], out_b[b]

            @plsc.parallel_loop(0, cnv, 1, unroll=_UNROLL)
            def body(i):
                x = lgb[pl.ds(i * _L, _L)]
                ci = jnp.minimum(ctb[pl.ds(i * _L, _L)], kmax)
                c = plsc.load_gather(tabr, [ci])
                e = jnp.exp(x * c)
                outb[pl.ds(i * _L, _L)] = pmv - p2mv / (e + 1.0)

            hout[j] = pltpu.async_copy(
                outb, out_hbm.at[pl.ds(base + j * cch, cch)], sout[b])
        hout[_NCHUNK - 2].wait()
        hout[_NCHUNK - 1].wait()

    return run


def _tc_body(par_ref, lg_ref, ct_ref, o_ref):
    x = lg_ref[...]
    idx = jnp.minimum(ct_ref[...], 10)
    g = jnp.zeros_like(x) + par_ref[0]
    for t in range(1, 11):
        g = jnp.where(idx >= t, par_ref[t], g)
    m = par_ref[16]
    o_ref[...] = m * jnp.tanh((0.5 * x) * g)


def kernel(logits, alt_counts, confidence, max_logit):
    n = logits.shape[0]
    k = confidence.shape[0]
    n_tc = _N_TC
    n_sc = n - n_tc

    m = max_logit.astype(jnp.float32)
    tab = jnp.zeros((_L,), jnp.float32).at[:k].set(confidence * (2.0 / m))
    params = jnp.concatenate(
        [tab, jnp.full((_L,), m, jnp.float32), jnp.full((_L,), 2.0 * m, jnp.float32)]
    )

    sc_run = _sc_run_factory(n, n_sc, k)
    sc_out = sc_run(logits, alt_counts, params)

    grid = n_tc // _BS_TC
    tc_full = pl.pallas_call(
        _tc_body,
        grid=(grid,),
        in_specs=[
            pl.BlockSpec((48,), lambda i: (0,)),
            pl.BlockSpec((_BS_TC,), lambda i: (i,)),
            pl.BlockSpec((_BS_TC,), lambda i: (i,)),
        ],
        out_specs=pl.BlockSpec((_BS_TC,), lambda i: (i,)),
        out_shape=jax.ShapeDtypeStruct((n,), jnp.float32),
    )(params, logits, alt_counts)

    return lax.dynamic_update_slice(tc_full, sc_out, (n_tc,))
